# baseline (speedup 1.0000x reference)
import functools

import jax
import jax.numpy as jnp
from jax import lax
from jax.experimental import pallas as pl
from jax.experimental.pallas import tpu as pltpu
from jax.experimental.pallas import tpu_sc as plsc

_VOCAB = 1000
_VPAD = 1024
_B, _L = 1024, 50
_N = _B * _L

_info = plsc.get_sparse_core_info()
_NC = _info.num_cores
_NS = _info.num_subcores
_NW = _NC * _NS
_BPW = _B // _NW
_IPW = _BPW * _L
_CH = 16
_NH = _BPW // _CH

_mesh = plsc.VectorSubcoreMesh(core_axis_name="c", subcore_axis_name="s")


@functools.partial(
    pl.kernel,
    mesh=_mesh,
    out_type=jax.ShapeDtypeStruct((_L * _B, 8, 128), jnp.float32),
    scratch_types=[
        pltpu.VMEM((_IPW,), jnp.int32),
        [pltpu.VMEM((_BPW, 8, 128), jnp.float32)] * 2,
        [pltpu.VMEM((_BPW,), jnp.int32)] * 2,
        [pltpu.SemaphoreType.DMA] * 2,
        [pltpu.SemaphoreType.DMA] * 2,
    ],
    compiler_params=pltpu.CompilerParams(needs_layout_passes=False),
)
def _gather_rows(idx_hbm, table_hbm, out_hbm, idx_v, bufs, ilst, semg, semw):
    wid = lax.axis_index("s") * _NC + lax.axis_index("c")
    b0 = wid * _BPW
    pltpu.sync_copy(idx_hbm.at[pl.ds(wid * _IPW, _IPW)], idx_v)
    lanes = lax.iota(jnp.int32, 16)

    def fill_idx(l, il):
        for h in (0, 1):
            offs = (h * 16 + lanes) * _L + l
            il[pl.ds(h * 16, 16)] = plsc.load_gather(idx_v, [offs])

    def gather(il, buf, sem):
        return pltpu.make_async_copy(table_hbm.at[il], buf, sem)

    def write(l, buf, sem):
        return pltpu.make_async_copy(
            buf, out_hbm.at[pl.ds(l * _B + b0, _BPW)], sem)

    fill_idx(0, ilst[0])
    gather(ilst[0], bufs[0], semg[0]).start()

    def body(l, carry):
        for p in (0, 1):
            gather(ilst[p], bufs[p], semg[p]).wait()

            @pl.when(l + p >= 1)
            def _():
                write(l + p - 1, bufs[1 - p], semw[1 - p]).wait()

            @pl.when(l + p + 1 < _L)
            def _():
                fill_idx(l + p + 1, ilst[1 - p])
                gather(ilst[1 - p], bufs[1 - p], semg[1 - p]).start()

            write(l + p, bufs[p], semw[p]).start()
        return carry

    lax.fori_loop(0, _L // 2, lambda i, c: body(i * 2, c), 0)
    write(_L - 1, bufs[1], semw[1]).wait()


def _transpose_body(x_ref, o_ref):
    x = x_ref[...]
    y = x.reshape(_B, _VPAD)
    z = y.T
    o_ref[...] = z[jnp.newaxis, :_VOCAB, :]


_transpose = pl.pallas_call(
    _transpose_body,
    grid=(_L,),
    in_specs=[pl.BlockSpec((_B, 8, 128), lambda l: (l, 0, 0))],
    out_specs=pl.BlockSpec((1, _VOCAB, _B), lambda l: (l, 0, 0)),
    out_shape=jax.ShapeDtypeStruct((_L, _VOCAB, _B), jnp.float32),
)


def kernel(idx, logits):
    table = jnp.pad(logits, ((0, 0), (0, _VPAD - _VOCAB)))
    rows = _gather_rows(idx.reshape(_N).astype(jnp.int32),
                        table.reshape(_VOCAB, 8, 128))
    out_t = _transpose(rows)
    return jnp.transpose(out_t, (2, 0, 1))
